# baseline (speedup 1.0000x reference)
import functools

import jax
import jax.numpy as jnp
from jax import lax
from jax.experimental import pallas as pl
from jax.experimental.pallas import tpu as pltpu
from jax.experimental.pallas import tpu_sc as plsc

_B, _L, _D, _V = 1024, 200, 64, 100000
_NC, _NS = 2, 16
_NW = _NC * _NS
_BPW = _B // _NW
_LA, _LB = 96, 104
_NK = _D // 16

_MB = 16


def _bag_kernel(x_hbm, emb_hbm, out_hbm, idx_v, rows_v, acc_v, sem0, sem1, sem2):
    wid = lax.axis_index("s") * _NC + lax.axis_index("c")
    base = wid * _BPW
    pltpu.sync_copy(x_hbm.at[pl.ds(base, _BPW)], idx_v)
    sems = (sem0, sem1)

    def issue(i, slot):
        return (
            pltpu.async_copy(emb_hbm.at[idx_v.at[i, pl.ds(0, _LA)]],
                             rows_v.at[slot, pl.ds(0, _LA)], sems[slot]),
            pltpu.async_copy(emb_hbm.at[idx_v.at[i, pl.ds(_LA, _LB)]],
                             rows_v.at[slot, pl.ds(_LA, _LB)], sems[slot]),
        )

    cps = [None, None]
    cps[0] = issue(0, 0)

    for i in range(_BPW):
        slot = i & 1
        if i + 1 < _BPW:
            cps[1 - slot] = issue(i + 1, 1 - slot)
        cps[slot][0].wait()
        cps[slot][1].wait()

        def red(t, acc):
            a = [acc[k] + rows_v[slot, 2 * t, pl.ds(16 * k, 16)]
                 for k in range(_NK)]
            b = [acc[_NK + k] + rows_v[slot, 2 * t + 1, pl.ds(16 * k, 16)]
                 for k in range(_NK)]
            return tuple(a + b)

        zeros = tuple(jnp.zeros((16,), jnp.float32) for _ in range(2 * _NK))
        acc = lax.fori_loop(0, _L // 2, red, zeros, unroll=4)
        for k in range(_NK):
            acc_v[i, pl.ds(16 * k, 16)] = acc[k] + acc[_NK + k]

    pltpu.async_copy(acc_v, out_hbm.at[pl.ds(base, _BPW)], sem2).wait()


def _embedding_bag(x_in, embedding_weight):
    mesh = plsc.VectorSubcoreMesh(core_axis_name="c", subcore_axis_name="s")
    k = functools.partial(
        pl.kernel,
        mesh=mesh,
        out_type=jax.ShapeDtypeStruct((_B, _D), jnp.float32),
        scratch_types=[
            pltpu.VMEM((_BPW, _L), jnp.int32),
            pltpu.VMEM((2, _L, _D), jnp.float32),
            pltpu.VMEM((_BPW, _D), jnp.float32),
            pltpu.SemaphoreType.DMA,
            pltpu.SemaphoreType.DMA,
            pltpu.SemaphoreType.DMA,
        ],
        compiler_params=pltpu.CompilerParams(use_tc_tiling_on_sc=False),
    )(_bag_kernel)
    return k(x_in, embedding_weight)


def _mm_kernel(x_ref, w_ref, b_ref, o_ref):
    o_ref[...] = lax.dot_general(
        x_ref[...], w_ref[...], (((1,), (1,)), ((), ())),
        preferred_element_type=jnp.float32) + b_ref[...]


def _matmul(x_sum, fc1_weight, fc1_bias):
    bias2 = fc1_bias.reshape(1, _V)
    return pl.pallas_call(
        _mm_kernel,
        grid=(_B // _MB,),
        in_specs=[
            pl.BlockSpec((_MB, _D), lambda b: (b, 0)),
            pl.BlockSpec((_V, _D), lambda b: (0, 0)),
            pl.BlockSpec((1, _V), lambda b: (0, 0)),
        ],
        out_specs=pl.BlockSpec((_MB, _V), lambda b: (b, 0)),
        out_shape=jax.ShapeDtypeStruct((_B, _V), jnp.float32),
        compiler_params=pltpu.CompilerParams(vmem_limit_bytes=110 * 1024 * 1024),
    )(x_sum, fc1_weight, bias2)


def kernel(x_in, embedding_weight, fc1_weight, fc1_bias):
    x_sum = _embedding_bag(x_in, embedding_weight)
    return _matmul(x_sum, fc1_weight, fc1_bias)


# speedup vs baseline: 3.3386x; 3.3386x over previous
import functools

import jax
import jax.numpy as jnp
from jax import lax
from jax.experimental import pallas as pl
from jax.experimental.pallas import tpu as pltpu
from jax.experimental.pallas import tpu_sc as plsc

_B, _L, _D, _V = 1024, 200, 64, 100000
_NC, _NS = 2, 16
_NW = _NC * _NS
_BPW = _B // _NW
_LA, _LB = 96, 104
_NK = _D // 16

_VB = 2048


def _bag_kernel(x_hbm, emb_hbm, out_hbm, idx_v, rows_v, acc_v, sem0, sem1, sem2):
    wid = lax.axis_index("s") * _NC + lax.axis_index("c")
    base = wid * _BPW
    pltpu.sync_copy(x_hbm.at[pl.ds(base, _BPW)], idx_v)
    sems = (sem0, sem1)

    def issue(i, slot):
        return (
            pltpu.async_copy(emb_hbm.at[idx_v.at[i, pl.ds(0, _LA)]],
                             rows_v.at[slot, pl.ds(0, _LA)], sems[slot]),
            pltpu.async_copy(emb_hbm.at[idx_v.at[i, pl.ds(_LA, _LB)]],
                             rows_v.at[slot, pl.ds(_LA, _LB)], sems[slot]),
        )

    cps = [None, None]
    cps[0] = issue(0, 0)

    for i in range(_BPW):
        slot = i & 1
        if i + 1 < _BPW:
            cps[1 - slot] = issue(i + 1, 1 - slot)
        cps[slot][0].wait()
        cps[slot][1].wait()

        def red(t, acc):
            a = [acc[k] + rows_v[slot, 2 * t, pl.ds(16 * k, 16)]
                 for k in range(_NK)]
            b = [acc[_NK + k] + rows_v[slot, 2 * t + 1, pl.ds(16 * k, 16)]
                 for k in range(_NK)]
            return tuple(a + b)

        zeros = tuple(jnp.zeros((16,), jnp.float32) for _ in range(2 * _NK))
        acc = lax.fori_loop(0, _L // 2, red, zeros, unroll=4)
        for k in range(_NK):
            acc_v[i, pl.ds(16 * k, 16)] = acc[k] + acc[_NK + k]

    pltpu.async_copy(acc_v, out_hbm.at[pl.ds(base, _BPW)], sem2).wait()


def _embedding_bag(x_in, embedding_weight):
    mesh = plsc.VectorSubcoreMesh(core_axis_name="c", subcore_axis_name="s")
    k = functools.partial(
        pl.kernel,
        mesh=mesh,
        out_type=jax.ShapeDtypeStruct((_B, _D), jnp.float32),
        scratch_types=[
            pltpu.VMEM((_BPW, _L), jnp.int32),
            pltpu.VMEM((2, _L, _D), jnp.float32),
            pltpu.VMEM((_BPW, _D), jnp.float32),
            pltpu.SemaphoreType.DMA,
            pltpu.SemaphoreType.DMA,
            pltpu.SemaphoreType.DMA,
        ],
        compiler_params=pltpu.CompilerParams(use_tc_tiling_on_sc=False),
    )(_bag_kernel)
    return k(x_in, embedding_weight)


def _mm_kernel(wt_ref, x_ref, b_ref, o_ref):
    o_ref[...] = lax.dot_general(
        wt_ref[...], x_ref[...], (((0,), (1,)), ((), ())),
        preferred_element_type=jnp.float32) + b_ref[...]


def _matmul(x_sum, fc1_weight, fc1_bias):
    wt = fc1_weight.T
    bias2 = fc1_bias.reshape(_V, 1)
    zt = pl.pallas_call(
        _mm_kernel,
        grid=(pl.cdiv(_V, _VB),),
        in_specs=[
            pl.BlockSpec((_D, _VB), lambda v: (0, v)),
            pl.BlockSpec((_B, _D), lambda v: (0, 0)),
            pl.BlockSpec((_VB, 1), lambda v: (v, 0)),
        ],
        out_specs=pl.BlockSpec((_VB, _B), lambda v: (v, 0)),
        out_shape=jax.ShapeDtypeStruct((_V, _B), jnp.float32),
    )(wt, x_sum, bias2)
    return zt.T


def kernel(x_in, embedding_weight, fc1_weight, fc1_bias):
    x_sum = _embedding_bag(x_in, embedding_weight)
    return _matmul(x_sum, fc1_weight, fc1_bias)


# speedup vs baseline: 3.3919x; 1.0160x over previous
import functools

import jax
import jax.numpy as jnp
from jax import lax
from jax.experimental import pallas as pl
from jax.experimental.pallas import tpu as pltpu
from jax.experimental.pallas import tpu_sc as plsc

_B, _L, _D, _V = 1024, 200, 64, 100000
_NC, _NS = 2, 16
_NW = _NC * _NS
_BPW = _B // _NW
_LA, _LB = 96, 104
_NK = _D // 16

_VB = 4096


def _bag_kernel(x_hbm, emb_hbm, out_hbm, idx_v, rows_v, acc_v, sem0, sem1, sem2):
    wid = lax.axis_index("s") * _NC + lax.axis_index("c")
    base = wid * _BPW
    pltpu.sync_copy(x_hbm.at[pl.ds(base, _BPW)], idx_v)
    sems = (sem0, sem1)

    def issue(i, slot):
        return (
            pltpu.async_copy(emb_hbm.at[idx_v.at[i, pl.ds(0, _LA)]],
                             rows_v.at[slot, pl.ds(0, _LA)], sems[slot]),
            pltpu.async_copy(emb_hbm.at[idx_v.at[i, pl.ds(_LA, _LB)]],
                             rows_v.at[slot, pl.ds(_LA, _LB)], sems[slot]),
        )

    cps = [None, None]
    cps[0] = issue(0, 0)

    for i in range(_BPW):
        slot = i & 1
        if i + 1 < _BPW:
            cps[1 - slot] = issue(i + 1, 1 - slot)
        cps[slot][0].wait()
        cps[slot][1].wait()

        def red(t, acc):
            a = [acc[k] + rows_v[slot, 2 * t, pl.ds(16 * k, 16)]
                 for k in range(_NK)]
            b = [acc[_NK + k] + rows_v[slot, 2 * t + 1, pl.ds(16 * k, 16)]
                 for k in range(_NK)]
            return tuple(a + b)

        zeros = tuple(jnp.zeros((16,), jnp.float32) for _ in range(2 * _NK))
        acc = lax.fori_loop(0, _L // 2, red, zeros, unroll=4)
        for k in range(_NK):
            acc_v[i, pl.ds(16 * k, 16)] = acc[k] + acc[_NK + k]

    pltpu.async_copy(acc_v, out_hbm.at[pl.ds(base, _BPW)], sem2).wait()


def _embedding_bag(x_in, embedding_weight):
    mesh = plsc.VectorSubcoreMesh(core_axis_name="c", subcore_axis_name="s")
    k = functools.partial(
        pl.kernel,
        mesh=mesh,
        out_type=jax.ShapeDtypeStruct((_B, _D), jnp.float32),
        scratch_types=[
            pltpu.VMEM((_BPW, _L), jnp.int32),
            pltpu.VMEM((2, _L, _D), jnp.float32),
            pltpu.VMEM((_BPW, _D), jnp.float32),
            pltpu.SemaphoreType.DMA,
            pltpu.SemaphoreType.DMA,
            pltpu.SemaphoreType.DMA,
        ],
        compiler_params=pltpu.CompilerParams(use_tc_tiling_on_sc=False),
    )(_bag_kernel)
    return k(x_in, embedding_weight)


def _mm_kernel(wt_ref, x_ref, b_ref, o_ref):
    o_ref[...] = lax.dot_general(
        wt_ref[...], x_ref[...], (((0,), (1,)), ((), ())),
        preferred_element_type=jnp.float32) + b_ref[...]


def _matmul(x_sum, fc1_weight, fc1_bias):
    wt = fc1_weight.T
    bias2 = fc1_bias.reshape(_V, 1)
    zt = pl.pallas_call(
        _mm_kernel,
        grid=(pl.cdiv(_V, _VB),),
        in_specs=[
            pl.BlockSpec((_D, _VB), lambda v: (0, v)),
            pl.BlockSpec((_B, _D), lambda v: (0, 0)),
            pl.BlockSpec((_VB, 1), lambda v: (v, 0)),
        ],
        out_specs=pl.BlockSpec((_VB, _B), lambda v: (v, 0)),
        out_shape=jax.ShapeDtypeStruct((_V, _B), jnp.float32),
    )(wt, x_sum, bias2)
    return zt.T


def kernel(x_in, embedding_weight, fc1_weight, fc1_bias):
    x_sum = _embedding_bag(x_in, embedding_weight)
    return _matmul(x_sum, fc1_weight, fc1_bias)
